# baseline (speedup 1.0000x reference)
import functools

import jax
import jax.numpy as jnp
from jax import lax
from jax.experimental import pallas as pl
from jax.experimental.pallas import tpu as pltpu
from jax.experimental.pallas import tpu_sc as plsc

_C = 80
_NC, _NS = 2, 16
_NW = _NC * _NS
_LANES = 16


def _emb_lookup(x2, table, pe_ext, t_len):
    n_chunks, c_sz = x2.shape
    _, d = table.shape
    per_w = n_chunks // _NW
    mesh = plsc.VectorSubcoreMesh(core_axis_name="c", subcore_axis_name="s")

    @functools.partial(
        pl.kernel,
        mesh=mesh,
        compiler_params=pltpu.CompilerParams(use_tc_tiling_on_sc=False),
        out_type=jax.ShapeDtypeStruct((n_chunks, c_sz, d), jnp.float32),
        scratch_types=[
            pltpu.VMEM((per_w, c_sz), jnp.int32),
            pltpu.VMEM((c_sz, d), jnp.float32),
            pltpu.VMEM(pe_ext.shape, jnp.float32),
            pltpu.SemaphoreType.DMA,
        ],
    )
    def k(x_hbm, table_hbm, pe_hbm, out_hbm, idx_v, rows_v, pos_v, sem):
        wid = lax.axis_index("s") * _NC + lax.axis_index("c")
        base = wid * per_w
        pltpu.sync_copy(x_hbm.at[pl.ds(base, per_w)], idx_v)
        pltpu.sync_copy(pe_hbm, pos_v)

        def chunk_body(c, carry):
            pltpu.async_copy(table_hbm.at[idx_v.at[c]], rows_v, sem).wait()
            pbase = (c * c_sz) % t_len

            def row_body(r, carry2):
                for dd in range(d // _LANES):
                    sl = pl.ds(dd * _LANES, _LANES)
                    rows_v[r, sl] = rows_v[r, sl] + pos_v[pbase + r, sl]
                return carry2

            lax.fori_loop(0, c_sz, row_body, 0)
            pltpu.sync_copy(rows_v, out_hbm.at[base + c])
            return carry

        lax.fori_loop(0, per_w, chunk_body, 0)

    return k(x2, table, pe_ext)


def kernel(x, table, pos_emb):
    bx, tx = x.shape
    _, d = table.shape
    bt = bx * tx
    x2 = x.reshape(bt // _C, _C).astype(jnp.int32)
    pe = pos_emb[:tx]
    pe_ext = jnp.concatenate([pe, pe[:_C]], axis=0)
    out = _emb_lookup(x2, table, pe_ext, tx)
    return out.reshape(bx, tx, d)
